# baseline (speedup 1.0000x reference)
import functools

import jax
import jax.numpy as jnp
from jax import lax
from jax.experimental import pallas as pl
from jax.experimental.pallas import tpu as pltpu
from jax.experimental.pallas import tpu_sc as plsc

NC = 2
NS = 16
NW = NC * NS
L = 16
CHUNK = 512
GB = 128
NB = CHUNK // GB


@functools.lru_cache(maxsize=None)
def _build(T, B, V, D):
    assert D == 64 and B % NW == 0 and T % CHUNK == 0 and CHUNK % GB == 0
    bags_w = B // NW
    nk = D // L

    mesh = plsc.VectorSubcoreMesh(core_axis_name="c", subcore_axis_name="s")

    def sread(ref, i):
        return ref[pl.ds(i, L)][0]

    @functools.partial(
        pl.kernel,
        mesh=mesh,
        compiler_params=pltpu.CompilerParams(use_tc_tiling_on_sc=False),
        out_type=jax.ShapeDtypeStruct((B * D,), jnp.float32),
        scratch_types=[
            pltpu.VMEM((bags_w + 24,), jnp.int32),
            pltpu.VMEM((8, GB), jnp.int32),
            pltpu.VMEM((8, GB), jnp.int32),
            pltpu.VMEM((CHUNK, D), jnp.float32),
            pltpu.VMEM((CHUNK, D), jnp.float32),
            pltpu.VMEM((bags_w * D,), jnp.float32),
            pltpu.SemaphoreType.DMA,
            pltpu.SemaphoreType.DMA,
        ],
    )
    def emb(idx_hbm, offs_hbm, table_hbm, out_hbm,
            offs_v, idx_a, idx_b, rows_a, rows_b, out_v, sem_a, sem_b):
        wid = lax.axis_index("s") * NC + lax.axis_index("c")
        bag0 = wid * bags_w
        pltpu.sync_copy(offs_hbm.at[pl.ds(bag0, bags_w)], offs_v.at[pl.ds(0, bags_w)])
        @pl.when(wid < NW - 1)
        def _():
            pltpu.sync_copy(offs_hbm.at[pl.ds(bag0 + bags_w, 8)],
                            offs_v.at[pl.ds(bags_w, 8)])

        @pl.when(wid == NW - 1)
        def _():
            offs_v[pl.ds(bags_w, L)] = jnp.full((L,), T, jnp.int32)

        p0 = sread(offs_v, 0)
        p1 = sread(offs_v, bags_w)
        a0 = (p0 // 8) * 8
        nchunks = jnp.maximum((p1 - a0 + CHUNK - 1) // CHUNK, 1)
        trips = (nchunks + 1) // 2

        zero = jnp.zeros((L,), jnp.float32)

        def cbase(cc):
            return jnp.minimum(a0 + cc * CHUNK, T - CHUNK)

        def startg(cc, idx_v, rows_v, sem):
            base = cbase(cc)
            for j in range(NB):
                pltpu.sync_copy(idx_hbm.at[pl.ds(base + j * GB, GB)], idx_v.at[j])
            for j in range(NB):
                pltpu.async_copy(table_hbm.at[idx_v.at[j]],
                                 rows_v.at[pl.ds(j * GB, GB)], sem)

        def waitg(idx_v, rows_v, sem):
            for j in range(NB):
                pltpu.make_async_copy(table_hbm.at[idx_v.at[j]],
                                      rows_v.at[pl.ds(j * GB, GB)], sem).wait()

        def consume(cc, rows_v, state):
            b = state[0]
            g0 = a0 + cc * CHUNK
            gend = jnp.minimum(g0 + CHUNK, p1)
            base = cbase(cc)

            def sum_rows(lo, hi, accs):
                def row_body(r, accs):
                    lr = r - base
                    return tuple(
                        accs[k] + rows_v[lr, k * L:(k + 1) * L] for k in range(nk)
                    )
                return plsc.parallel_loop(lo, hi, carry=accs, unroll=4)(row_body)

            def bs_body(_, lohi):
                lo, hi = lohi
                mid = (lo + hi + 1) // 2
                take = sread(offs_v, mid) <= gend
                return (jnp.where(take, mid, lo), jnp.where(take, hi, mid - 1))

            b_end, _ = lax.fori_loop(0, 10, bs_body, (b, jnp.int32(bags_w)))

            def bag_body(b, carry):
                s = carry[0]
                accs = carry[1:]
                e = sread(offs_v, b + 1)
                accs = sum_rows(jnp.maximum(s, g0), e, accs)
                cntv = jnp.full((L,), jnp.maximum(e - s, 1))
                sc = 1.0 / cntv.astype(jnp.float32)
                for k in range(nk):
                    out_v[pl.ds(b * D + k * L, L)] = accs[k] * sc
                return (e,) + (zero,) * nk

            s0 = sread(offs_v, b)
            st = lax.fori_loop(b, b_end, bag_body, (s0,) + state[1:])
            lo = jnp.minimum(jnp.maximum(st[0], g0), gend)
            accs = sum_rows(lo, gend, st[1:])
            return (b_end,) + accs

        startg(0, idx_a, rows_a, sem_a)

        def pipe_body(i, state):
            cc = 2 * i
            waitg(idx_a, rows_a, sem_a)
            startg(cc + 1, idx_b, rows_b, sem_b)
            state = consume(cc, rows_a, state)
            waitg(idx_b, rows_b, sem_b)
            startg(cc + 2, idx_a, rows_a, sem_a)
            state = consume(cc + 1, rows_b, state)
            return state

        lax.fori_loop(0, trips, pipe_body, (jnp.int32(0),) + (zero,) * nk)
        waitg(idx_a, rows_a, sem_a)
        pltpu.sync_copy(out_v, out_hbm.at[pl.ds(bag0 * D, bags_w * D)])

    return emb


def kernel(subword_idx, offsets, table):
    T = subword_idx.shape[0]
    B = offsets.shape[0]
    V, D = table.shape
    emb = _build(T, B, V, D)
    out = emb(subword_idx, offsets, table)
    return out.reshape(B, D)


# speedup vs baseline: 1.1372x; 1.1372x over previous
import functools

import jax
import jax.numpy as jnp
from jax import lax
from jax.experimental import pallas as pl
from jax.experimental.pallas import tpu as pltpu
from jax.experimental.pallas import tpu_sc as plsc

NC = 2
NS = 16
NW = NC * NS
L = 16
CHUNK = 256
GB = 128
NB = CHUNK // GB


@functools.lru_cache(maxsize=None)
def _build(T, B, V, D):
    assert D == 64 and B % NW == 0 and T % CHUNK == 0 and CHUNK % GB == 0
    bags_w = B // NW
    nk = D // L

    mesh = plsc.VectorSubcoreMesh(core_axis_name="c", subcore_axis_name="s")

    def sread(ref, i):
        return ref[pl.ds(i, L)][0]

    @functools.partial(
        pl.kernel,
        mesh=mesh,
        out_type=jax.ShapeDtypeStruct((B * D,), jnp.float32),
        scratch_types=[
            pltpu.VMEM((bags_w + 24,), jnp.int32),
            pltpu.VMEM((8, GB), jnp.int32),
            pltpu.VMEM((8, GB), jnp.int32),
            pltpu.VMEM((CHUNK, 2 * D), jnp.float32),
            pltpu.VMEM((CHUNK, 2 * D), jnp.float32),
            pltpu.VMEM((bags_w * D,), jnp.float32),
            pltpu.SemaphoreType.DMA,
            pltpu.SemaphoreType.DMA,
            pltpu.SemaphoreType.DMA,
            pltpu.SemaphoreType.DMA,
        ],
    )
    def emb(idx_hbm, offs_hbm, table_hbm, out_hbm,
            offs_v, idx_a, idx_b, rows_a, rows_b, out_v,
            sem_a, sem_b, isem_a, isem_b):
        wid = lax.axis_index("s") * NC + lax.axis_index("c")
        bag0 = wid * bags_w
        pltpu.sync_copy(offs_hbm.at[pl.ds(bag0, bags_w)], offs_v.at[pl.ds(0, bags_w)])
        @pl.when(wid < NW - 1)
        def _():
            pltpu.sync_copy(offs_hbm.at[pl.ds(bag0 + bags_w, 8)],
                            offs_v.at[pl.ds(bags_w, 8)])

        @pl.when(wid == NW - 1)
        def _():
            offs_v[pl.ds(bags_w, L)] = jnp.full((L,), T, jnp.int32)

        p0 = sread(offs_v, 0)
        p1 = sread(offs_v, bags_w)
        a0 = (p0 // 8) * 8
        nchunks = jnp.maximum((p1 - a0 + CHUNK - 1) // CHUNK, 1)
        trips = (nchunks + 1) // 2

        zero = jnp.zeros((L,), jnp.float32)

        def cbase(cc):
            return jnp.minimum(a0 + cc * CHUNK, T - CHUNK)

        def start_idx(cc, idx_v, isem):
            base = cbase(cc)
            for j in range(NB):
                pltpu.async_copy(idx_hbm.at[pl.ds(base + j * GB, GB)],
                                 idx_v.at[j], isem)

        def wait_idx(cc, idx_v, isem):
            base = cbase(cc)
            for j in range(NB):
                pltpu.make_async_copy(idx_hbm.at[pl.ds(base + j * GB, GB)],
                                      idx_v.at[j], isem).wait()

        def fire_gather(idx_v, rows_v, sem):
            for j in range(NB):
                pltpu.async_copy(table_hbm.at[idx_v.at[j]],
                                 rows_v.at[pl.ds(j * GB, GB)], sem)

        def waitg(idx_v, rows_v, sem):
            for j in range(NB):
                pltpu.make_async_copy(table_hbm.at[idx_v.at[j]],
                                      rows_v.at[pl.ds(j * GB, GB)], sem).wait()

        def consume(cc, rows_v, state):
            b = state[0]
            g0 = a0 + cc * CHUNK
            gend = jnp.minimum(g0 + CHUNK, p1)
            base = cbase(cc)

            def sum_rows(lo, hi, accs):
                def row_body(r, accs):
                    lr = r - base
                    return tuple(
                        accs[k] + rows_v[lr, k * L:(k + 1) * L] for k in range(nk)
                    )
                return plsc.parallel_loop(lo, hi, carry=accs, unroll=4)(row_body)

            def bs_body(_, lohi):
                lo, hi = lohi
                mid = (lo + hi + 1) // 2
                take = sread(offs_v, mid) <= gend
                return (jnp.where(take, mid, lo), jnp.where(take, hi, mid - 1))

            b_end, _ = lax.fori_loop(0, 10, bs_body, (b, jnp.int32(bags_w)))

            def bag_body(b, carry):
                s = carry[0]
                accs = carry[1:]
                e = sread(offs_v, b + 1)
                accs = sum_rows(jnp.maximum(s, g0), e, accs)
                cntv = jnp.full((L,), jnp.maximum(e - s, 1))
                sc = 1.0 / cntv.astype(jnp.float32)
                for k in range(nk):
                    out_v[pl.ds(b * D + k * L, L)] = accs[k] * sc
                return (e,) + (zero,) * nk

            s0 = sread(offs_v, b)
            st = lax.fori_loop(b, b_end, bag_body, (s0,) + state[1:])
            lo = jnp.minimum(jnp.maximum(st[0], g0), gend)
            accs = sum_rows(lo, gend, st[1:])
            return (b_end,) + accs

        start_idx(0, idx_a, isem_a)
        wait_idx(0, idx_a, isem_a)
        fire_gather(idx_a, rows_a, sem_a)
        start_idx(1, idx_b, isem_b)

        def pipe_body(i, state):
            cc = 2 * i
            waitg(idx_a, rows_a, sem_a)
            wait_idx(cc + 1, idx_b, isem_b)
            fire_gather(idx_b, rows_b, sem_b)
            start_idx(cc + 2, idx_a, isem_a)
            state = consume(cc, rows_a, state)
            waitg(idx_b, rows_b, sem_b)
            wait_idx(cc + 2, idx_a, isem_a)
            fire_gather(idx_a, rows_a, sem_a)
            start_idx(cc + 3, idx_b, isem_b)
            state = consume(cc + 1, rows_b, state)
            return state

        lax.fori_loop(0, trips, pipe_body, (jnp.int32(0),) + (zero,) * nk)
        waitg(idx_a, rows_a, sem_a)
        wait_idx(2 * trips + 1, idx_b, isem_b)
        pltpu.sync_copy(out_v, out_hbm.at[pl.ds(bag0 * D, bags_w * D)])

    return emb


def kernel(subword_idx, offsets, table):
    T = subword_idx.shape[0]
    B = offsets.shape[0]
    V, D = table.shape
    emb = _build(T, B, V, D)
    table128 = jnp.pad(table, ((0, 0), (0, 2 * D - D)))
    out = emb(subword_idx, offsets, table128)
    return out.reshape(B, D)
